# baseline (speedup 1.0000x reference)
import functools

import jax
import jax.numpy as jnp
from jax import lax
from jax.experimental import pallas as pl
from jax.experimental.pallas import tpu as pltpu
from jax.experimental.pallas import tpu_sc as plsc

B = 4096
H = 50
D = 64
NC = 2
NS = 16
NW = NC * NS
BAGS_PER_W = B // NW
LANES = 16
DCH = D // LANES
RG = 10
NG = H // RG
CHUNK = 4
NCHUNK = BAGS_PER_W // CHUNK
NBUF = 4


def _bag_sums(idx2, emb):
    mesh = plsc.VectorSubcoreMesh(core_axis_name="c", subcore_axis_name="s")

    @functools.partial(
        pl.kernel,
        out_type=jax.ShapeDtypeStruct((B, D), jnp.float32),
        mesh=mesh,
        scratch_types=[
            pltpu.VMEM((NCHUNK, CHUNK * H), jnp.int32),
            pltpu.VMEM((CHUNK * H, D), jnp.float32),
            pltpu.VMEM((CHUNK * H, D), jnp.float32),
            pltpu.VMEM((CHUNK * H, D), jnp.float32),
            pltpu.VMEM((CHUNK * H, D), jnp.float32),
            pltpu.VMEM((BAGS_PER_W, D), jnp.float32),
            pltpu.SemaphoreType.DMA,
            pltpu.SemaphoreType.DMA,
            pltpu.SemaphoreType.DMA,
            pltpu.SemaphoreType.DMA,
        ],
        compiler_params=pltpu.CompilerParams(use_tc_tiling_on_sc=False),
    )
    def k(idx_hbm, emb_hbm, out_hbm, idx_v, b0, b1, b2, b3, out_v,
          s0, s1, s2, s3):
        wid = lax.axis_index("s") * NC + lax.axis_index("c")
        pltpu.sync_copy(idx_hbm.at[pl.ds(wid * NCHUNK, NCHUNK)], idx_v)
        bufs = (b0, b1, b2, b3)
        sems = (s0, s1, s2, s3)

        def off(c):
            return idx_v.at[c]

        for c in range(NBUF - 1):
            pltpu.async_copy(emb_hbm.at[off(c)], bufs[c], sems[c])

        def group(g, carry):
            for bslot in range(NBUF):
                c = g * NBUF + bslot
                pltpu.make_async_copy(
                    emb_hbm.at[off(c)], bufs[bslot], sems[bslot]
                ).wait()

                @pl.when(c + NBUF - 1 < NCHUNK)
                def _():
                    nxt = (bslot + NBUF - 1) % NBUF
                    pltpu.async_copy(
                        emb_hbm.at[off(c + NBUF - 1)], bufs[nxt], sems[nxt]
                    )

                for p in range(CHUNK):
                    def body(r, acc, _b=bslot, _p=p):
                        base = _p * H + r * RG
                        for rr in range(RG):
                            acc = tuple(
                                acc[ch]
                                + bufs[_b][base + rr,
                                           pl.ds(ch * LANES, LANES)]
                                for ch in range(DCH)
                            )
                        return acc

                    acc = lax.fori_loop(
                        0, NG, body,
                        tuple(
                            jnp.zeros((LANES,), jnp.float32)
                            for _ in range(DCH)
                        ),
                    )
                    bag = c * CHUNK + p
                    for ch in range(DCH):
                        out_v[bag, pl.ds(ch * LANES, LANES)] = acc[ch]
            return carry

        lax.fori_loop(0, NCHUNK // NBUF, group, 0)
        pltpu.sync_copy(
            out_v, out_hbm.at[pl.ds(wid * BAGS_PER_W, BAGS_PER_W)]
        )

    return k(idx2, emb)


def _head(x, w, bias):
    alpha = 1.6732632423543772
    scale = 1.0507009873554805

    def body(x_ref, w_ref, b_ref, o_ref):
        xv = x_ref[...]
        xv = scale * jnp.where(xv > 0, xv, alpha * (jnp.exp(xv) - 1.0))
        o_ref[...] = (
            lax.dot_general(
                xv, w_ref[...], (((1,), (1,)), ((), ())),
                preferred_element_type=jnp.float32,
            )
            + b_ref[...]
        )

    return pl.pallas_call(
        body,
        out_shape=jax.ShapeDtypeStruct((B, D), jnp.float32),
    )(x, w, bias)


def kernel(input_text, emb, W, b):
    idx2 = input_text.astype(jnp.int32).reshape(B // CHUNK, CHUNK * H)
    sums = _bag_sums(idx2, emb)
    return _head(sums, W, b.reshape(1, D))
